# baseline (speedup 1.0000x reference)
import jax
import jax.numpy as jnp
from jax import lax
from jax.experimental import pallas as pl
from jax.experimental.pallas import tpu as pltpu

_BM = 448


def _gcn_body(input_ref, weight_ref, adj_ref, out_ref, support_ref):
    @pl.when(pl.program_id(0) == 0)
    def _compute_support():
        support_ref[...] = jnp.dot(
            input_ref[...], weight_ref[...],
            precision=lax.Precision.DEFAULT,
            preferred_element_type=jnp.float32,
        )

    acc = jnp.dot(
        adj_ref[...], support_ref[...],
        precision=lax.Precision.DEFAULT,
        preferred_element_type=jnp.float32,
    )
    out_ref[...] = jnp.maximum(acc, 0.0)


def kernel(input, adj, weight):
    n, d_in = input.shape
    d_out = weight.shape[1]
    return pl.pallas_call(
        _gcn_body,
        grid=(pl.cdiv(n, _BM),),
        in_specs=[
            pl.BlockSpec((n, d_in), lambda i: (0, 0)),
            pl.BlockSpec((d_in, d_out), lambda i: (0, 0)),
            pl.BlockSpec((_BM, n), lambda i: (i, 0)),
        ],
        out_specs=pl.BlockSpec((_BM, d_out), lambda i: (i, 0)),
        out_shape=jax.ShapeDtypeStruct((n, d_out), jnp.float32),
        scratch_shapes=[pltpu.VMEM((n, d_out), jnp.float32)],
    )(input.astype(jnp.float32), weight, adj)
